# baseline (speedup 1.0000x reference)
import jax
import jax.numpy as jnp
from jax.experimental import pallas as pl


def _head_body(cat_ref, mean_ref, var_ref, gamma_ref, beta_ref, wf_ref, bf_ref, o_ref):
    cat = cat_ref[...]
    mean = mean_ref[...]
    var = var_ref[...]
    bn = (cat - mean) / jnp.sqrt(var + 1e-5) * gamma_ref[...] + beta_ref[...]
    acc = jax.lax.dot_general(
        bn, wf_ref[...], (((1,), (1,)), ((), ())),
        preferred_element_type=jnp.float32)
    o_ref[...] = acc + bf_ref[...]


def _head(cat, mean, var, gamma, beta, W_final, b_final):
    n, c = cat.shape
    out_dim = W_final.shape[0]
    blk = 1000
    grid = (n // blk,)
    return pl.pallas_call(
        _head_body,
        grid=grid,
        in_specs=[
            pl.BlockSpec((blk, c), lambda i: (i, 0)),
            pl.BlockSpec((c,), lambda i: (0,)),
            pl.BlockSpec((c,), lambda i: (0,)),
            pl.BlockSpec((c,), lambda i: (0,)),
            pl.BlockSpec((c,), lambda i: (0,)),
            pl.BlockSpec((out_dim, c), lambda i: (0, 0)),
            pl.BlockSpec((out_dim,), lambda i: (0,)),
        ],
        out_specs=pl.BlockSpec((blk, out_dim), lambda i: (i, 0)),
        out_shape=jax.ShapeDtypeStruct((n, out_dim), jnp.float32),
    )(cat, mean, var, gamma, beta, W_final, b_final)


def kernel(x, ei, n_nodes, W_proj, b_proj, W_layers, b_layers, W_aggr, b_aggr,
           bn_gamma, bn_beta, W_final, b_final):
    N = x.shape[0]
    src = ei[0]
    dst = ei[1]
    h = jax.nn.relu(x @ W_proj.T + b_proj)
    L = W_layers.shape[0]
    u = h
    for i in range(L):
        x_i = h @ W_layers[i].T + b_layers[i]
        u_i = h @ W_aggr[i].T + b_aggr[i]
        u = jax.ops.segment_sum(u_i[src], dst, num_segments=N)
        h = jax.nn.relu(x_i + u)
    ns = x.shape[0]
    num_graphs = N // ns
    u_g = u.reshape(num_graphs, ns, -1).sum(axis=1)
    u_g = jnp.repeat(u_g, ns, axis=0)
    cat = jnp.concatenate([h, u_g], axis=1)
    mean = cat.mean(axis=0)
    var = cat.var(axis=0)
    out = _head(cat, mean, var, bn_gamma, bn_beta, W_final, b_final)
    out = out * jnp.asarray(n_nodes // ns, out.dtype)
    return out.reshape(num_graphs, -1)


# speedup vs baseline: 1.0607x; 1.0607x over previous
import jax
import jax.numpy as jnp
from jax.experimental import pallas as pl

_BLK = 1000


def _mm_pair_first_body(x_ref, wp_ref, bp_ref, wl_ref, bl_ref, wa_ref, ba_ref,
                        xi_ref, ui_ref):
    x = x_ref[...]
    h = jax.nn.relu(
        jax.lax.dot_general(x, wp_ref[...], (((1,), (1,)), ((), ())),
                            preferred_element_type=jnp.float32) + bp_ref[...])
    xi_ref[...] = jax.lax.dot_general(
        h, wl_ref[...], (((1,), (1,)), ((), ())),
        preferred_element_type=jnp.float32) + bl_ref[...]
    ui_ref[...] = jax.lax.dot_general(
        h, wa_ref[...], (((1,), (1,)), ((), ())),
        preferred_element_type=jnp.float32) + ba_ref[...]


def _mm_pair_next_body(xp_ref, up_ref, wl_ref, bl_ref, wa_ref, ba_ref,
                       xi_ref, ui_ref):
    h = jax.nn.relu(xp_ref[...] + up_ref[...])
    xi_ref[...] = jax.lax.dot_general(
        h, wl_ref[...], (((1,), (1,)), ((), ())),
        preferred_element_type=jnp.float32) + bl_ref[...]
    ui_ref[...] = jax.lax.dot_general(
        h, wa_ref[...], (((1,), (1,)), ((), ())),
        preferred_element_type=jnp.float32) + ba_ref[...]


def _row_spec(d):
    return pl.BlockSpec((_BLK, d), lambda i: (i, 0))


def _w_spec(a, b):
    return pl.BlockSpec((a, b), lambda i: (0, 0))


def _b_spec(d):
    return pl.BlockSpec((d,), lambda i: (0,))


def _mm_pair_first(x, Wp, bp, Wl, bl, Wa, ba):
    n, d_in = x.shape
    hid = Wl.shape[0]
    return pl.pallas_call(
        _mm_pair_first_body,
        grid=(n // _BLK,),
        in_specs=[_row_spec(d_in), _w_spec(hid, d_in), _b_spec(hid),
                  _w_spec(hid, hid), _b_spec(hid),
                  _w_spec(hid, hid), _b_spec(hid)],
        out_specs=[_row_spec(hid), _row_spec(hid)],
        out_shape=[jax.ShapeDtypeStruct((n, hid), jnp.float32),
                   jax.ShapeDtypeStruct((n, hid), jnp.float32)],
    )(x, Wp, bp, Wl, bl, Wa, ba)


def _mm_pair_next(xp, up, Wl, bl, Wa, ba):
    n, hid = xp.shape
    return pl.pallas_call(
        _mm_pair_next_body,
        grid=(n // _BLK,),
        in_specs=[_row_spec(hid), _row_spec(hid),
                  _w_spec(hid, hid), _b_spec(hid),
                  _w_spec(hid, hid), _b_spec(hid)],
        out_specs=[_row_spec(hid), _row_spec(hid)],
        out_shape=[jax.ShapeDtypeStruct((n, hid), jnp.float32),
                   jax.ShapeDtypeStruct((n, hid), jnp.float32)],
    )(xp, up, Wl, bl, Wa, ba)


def _head_body(xp_ref, up_ref, mean_ref, var_ref, gamma_ref, beta_ref,
               ug_ref, wf_ref, bf_ref, o_ref):
    h = jax.nn.relu(xp_ref[...] + up_ref[...])
    hid = h.shape[1]
    mean = mean_ref[...]
    var = var_ref[...]
    gamma = gamma_ref[...]
    beta = beta_ref[...]
    denom = jnp.sqrt(var + 1e-5)
    bn1 = (h - mean[:hid]) / denom[:hid] * gamma[:hid] + beta[:hid]
    ug = ug_ref[...]
    bn2_row = (ug - mean[hid:]) / denom[hid:] * gamma[hid:] + beta[hid:]
    bn2 = jnp.broadcast_to(bn2_row.reshape(1, hid), h.shape)
    acc = jax.lax.dot_general(
        bn1, wf_ref[..., :hid], (((1,), (1,)), ((), ())),
        preferred_element_type=jnp.float32)
    acc = acc + jax.lax.dot_general(
        bn2, wf_ref[..., hid:], (((1,), (1,)), ((), ())),
        preferred_element_type=jnp.float32)
    o_ref[...] = acc + bf_ref[...]


def _head(xp, up, mean, var, gamma, beta, u_g, W_final, b_final):
    n, hid = xp.shape
    out_dim, c = W_final.shape
    return pl.pallas_call(
        _head_body,
        grid=(n // _BLK,),
        in_specs=[_row_spec(hid), _row_spec(hid),
                  _b_spec(c), _b_spec(c), _b_spec(c), _b_spec(c),
                  _b_spec(hid), _w_spec(out_dim, c), _b_spec(out_dim)],
        out_specs=_row_spec(out_dim),
        out_shape=jax.ShapeDtypeStruct((n, out_dim), jnp.float32),
    )(xp, up, mean, var, gamma, beta, u_g, W_final, b_final)


def kernel(x, ei, n_nodes, W_proj, b_proj, W_layers, b_layers, W_aggr, b_aggr,
           bn_gamma, bn_beta, W_final, b_final):
    N = x.shape[0]
    src = ei[0]
    dst = ei[1]

    x_0, u_0 = _mm_pair_first(x, W_proj, b_proj,
                              W_layers[0], b_layers[0], W_aggr[0], b_aggr[0])
    u1 = jax.ops.segment_sum(u_0[src], dst, num_segments=N)
    x_1, u_1 = _mm_pair_next(x_0, u1, W_layers[1], b_layers[1],
                             W_aggr[1], b_aggr[1])
    u2 = jax.ops.segment_sum(u_1[src], dst, num_segments=N)

    ns = x.shape[0]
    num_graphs = N // ns
    h2 = jax.nn.relu(x_1 + u2)
    u_g = u2.reshape(num_graphs, ns, -1).sum(axis=1)
    u_rep = jnp.repeat(u_g, ns, axis=0)
    cat = jnp.concatenate([h2, u_rep], axis=1)
    mean = cat.mean(axis=0)
    var = cat.var(axis=0)

    out = _head(x_1, u2, mean, var, bn_gamma, bn_beta, u_g.reshape(-1),
                W_final, b_final)
    out = out * jnp.asarray(n_nodes // ns, out.dtype)
    return out.reshape(num_graphs, -1)
